# baseline (speedup 1.0000x reference)
import jax
import jax.numpy as jnp
from jax import lax
from jax.experimental import pallas as pl
from jax.experimental.pallas import tpu as pltpu

_BI = 1024


def _rgcn_body(X_ref, A_ref, Wl_ref, Win_ref, Wout_ref, out_ref,
               xT, hin_augs, houtT_augs, y2t_augs):
    r = pl.program_id(0)
    i = pl.program_id(1)
    nr = pl.num_programs(0)
    ni = pl.num_programs(1)
    n, od = out_ref.shape
    bi = A_ref.shape[1]
    aug_w = hin_augs.shape[2]
    aug_h = houtT_augs.shape[1]

    @pl.when(jnp.logical_and(r == 0, i == 0))
    def _init():
        out_ref[...] = jnp.dot(X_ref[...], Wl_ref[...],
                               preferred_element_type=jnp.float32)
        xT[...] = X_ref[...].T
        y2t_augs[...] = jnp.zeros_like(y2t_augs)
        for rel in range(nr):
            cid = lax.broadcasted_iota(jnp.int32, (n, aug_w - od), 1)
            hin_augs[rel, :, od:] = jnp.where(
                cid == 0, 1.0, 0.0).astype(jnp.bfloat16)
            rid = lax.broadcasted_iota(jnp.int32, (aug_h - od, n), 0)
            houtT_augs[rel, od:, :] = jnp.where(
                rid == 0, 1.0, 0.0).astype(jnp.bfloat16)
            hin_augs[rel, :, :od] = jnp.dot(
                X_ref[...], Win_ref[rel],
                preferred_element_type=jnp.float32).astype(jnp.bfloat16)
            houtT_augs[rel, :od, :] = lax.dot_general(
                Wout_ref[rel], xT[...], (((0,), (0,)), ((), ())),
                preferred_element_type=jnp.float32).astype(jnp.bfloat16)

    def _finish_relation(rel):
        y2t = y2t_augs[pl.ds(rel, 1)][0]
        inv_c = 1.0 / jnp.maximum(y2t[od:od + 1, :], 1e-12)
        out_ref[...] += (y2t[:od, :] * inv_c).T

    @pl.when(jnp.logical_and(r > 0, i == 0))
    def _finish_prev():
        _finish_relation(r - 1)

    a = A_ref[0].astype(jnp.bfloat16)
    hin_r = hin_augs[pl.ds(r, 1)][0]
    p = jnp.dot(a, hin_r, preferred_element_type=jnp.float32)
    inv_r = 1.0 / jnp.maximum(p[:, od:od + 1], 1e-12)
    out_ref[pl.ds(i * bi, bi), :] += p[:, :od] * inv_r
    houtT_r = houtT_augs[pl.ds(r, 1), :, pl.ds(i * bi, bi)][0]
    y2t_augs[pl.ds(r, 1)] += jnp.dot(
        houtT_r, a, preferred_element_type=jnp.float32)[None]

    @pl.when(jnp.logical_and(r == nr - 1, i == ni - 1))
    def _finish_last():
        _finish_relation(nr - 1)


def kernel(X, A, W_loop, W_in, W_out):
    n, in_dim = X.shape
    r_count = A.shape[0]
    out_dim = W_loop.shape[1]
    bi = min(_BI, n)
    ni = n // bi
    aug_w = max(128, out_dim + 1)
    aug_h = out_dim + 8

    return pl.pallas_call(
        _rgcn_body,
        grid=(r_count, ni),
        in_specs=[
            pl.BlockSpec((n, in_dim), lambda r, i: (0, 0)),
            pl.BlockSpec((1, bi, n), lambda r, i: (r, i, 0)),
            pl.BlockSpec((in_dim, out_dim), lambda r, i: (0, 0)),
            pl.BlockSpec((r_count, in_dim, out_dim), lambda r, i: (0, 0, 0)),
            pl.BlockSpec((r_count, in_dim, out_dim), lambda r, i: (0, 0, 0)),
        ],
        out_specs=pl.BlockSpec((n, out_dim), lambda r, i: (0, 0)),
        out_shape=jax.ShapeDtypeStruct((n, out_dim), jnp.float32),
        scratch_shapes=[
            pltpu.VMEM((in_dim, n), jnp.float32),
            pltpu.VMEM((r_count, n, aug_w), jnp.bfloat16),
            pltpu.VMEM((r_count, aug_h, n), jnp.bfloat16),
            pltpu.VMEM((r_count, aug_h, n), jnp.float32),
        ],
    )(X, A, W_loop, W_in, W_out)
